# baseline (speedup 1.0000x reference)
import functools

import jax
import jax.numpy as jnp
from jax import lax
from jax.experimental import pallas as pl
from jax.experimental.pallas import tpu as pltpu
from jax.experimental.pallas import tpu_sc as plsc

P = 16
F = 128
N = 10000
NA = 10240
E = 320000
NLINK = 100000
EDGE_MLP = 32

NC = 2
NS = 16
NW = NC * NS
LNS = 16

CH = 128
ECHUNKS = 80
EPT = CH * ECHUNKS
SCH = 64
SECHUNKS = EPT // SCH
SE0 = 10240
SE1 = 10240
SQ0 = SE0 // SCH // 4
SQ1 = SE1 // SCH // 4
LL0 = 4608
LL1 = 1792
EPAD = EPT * NW
RPT = NA // NS

LCH = 128
LCHUNKS = 25
LPT = LCH * LCHUNKS
LPAD = LPT * NW

GB = 2048
GBLKS = EPAD // GB
RB = 1000

_MESH = plsc.VectorSubcoreMesh(
    core_axis_name="c", subcore_axis_name="s", num_cores=NC, num_subcores=NS)


@functools.partial(
    pl.kernel,
    out_type=(
        jax.ShapeDtypeStruct((NC, NA, LNS), jnp.float32),
        jax.ShapeDtypeStruct((EPAD,), jnp.float32),
        jax.ShapeDtypeStruct((EPAD,), jnp.float32),
    ),
    mesh=_MESH,
    compiler_params=pltpu.CompilerParams(use_tc_tiling_on_sc=False, needs_layout_passes=False),
    scratch_types=(
        [pltpu.VMEM((CH,), jnp.int32)] * 2 +
        [pltpu.VMEM((CH,), jnp.int32)] * 4 +
        [pltpu.VMEM((CH, P), jnp.float32)] * 2 +
        [pltpu.VMEM((CH, P), jnp.float32)] * 2 +
        [pltpu.VMEM((CH,), jnp.float32)] * 2 +
        [pltpu.VMEM((CH,), jnp.float32)] * 2 +
        [pltpu.VMEM((CH, LNS), jnp.float32)] +
        [pltpu.VMEM((LNS,), jnp.float32)] * 4 +
        [pltpu.VMEM_SHARED((NA, LNS), jnp.float32)] +
        [pltpu.SemaphoreType.DMA] * 16
    ),
)
def _edge_prep(row_hbm, col_hbm, coors_hbm, ones_hbm, zeros_hbm,
               a1_hbm, c1_hbm, a2_hbm, c2_hbm,
               deg_out, g1_out, g2_out,
               rv0, rv1, cv0, cv1, cv2, cv3, br0, br1, bc0, bc1,
               o10, o11, o20, o21, ones_v, a1v, c1v, a2v, c2v, acc,
               smr0, smr1, smc0, smc1, smc2, smc3,
               smgr0, smgr1, smgc0, smgc1,
               smo10, smo11, smo20, smo21, smd0, smd1):
    cid = lax.axis_index("c")
    sid = lax.axis_index("s")
    wid = sid * NC + cid
    ebase = wid * EPT
    rowv = [rv0, rv1]
    colv = [cv0, cv1, cv2, cv3]
    br = [br0, br1]
    bc = [bc0, bc1]
    og1 = [o10, o11]
    og2 = [o20, o21]
    smr = [smr0, smr1]
    smc = [smc0, smc1, smc2, smc3]
    smgr = [smgr0, smgr1]
    smgc = [smgc0, smgc1]
    smo1 = [smo10, smo11]
    smo2 = [smo20, smo21]
    smd = [smd0, smd1]

    def issue_row(j, s):
        pltpu.async_copy(row_hbm.at[pl.ds(ebase + j * CH, CH)], rowv[s], smr[s])

    def issue_col(j, s):
        pltpu.async_copy(col_hbm.at[pl.ds(ebase + j * CH, CH)], colv[s], smc[s])

    def wait_row(s):
        pltpu.make_async_copy(row_hbm.at[pl.ds(0, CH)], rowv[s], smr[s]).wait()

    def wait_col(s):
        pltpu.make_async_copy(col_hbm.at[pl.ds(0, CH)], colv[s], smc[s]).wait()

    def issue_gr(s):
        pltpu.async_copy(coors_hbm.at[rowv[s]], br[s], smgr[s])

    def issue_gc(s, cs):
        pltpu.async_copy(coors_hbm.at[colv[cs]], bc[s], smgc[s])

    def wait_gr(s):
        pltpu.make_async_copy(coors_hbm.at[rowv[s]], br[s], smgr[s]).wait()

    def wait_gc(s, cs):
        pltpu.make_async_copy(coors_hbm.at[colv[cs]], bc[s], smgc[s]).wait()

    def issue_out(j, s):
        pltpu.async_copy(og1[s], g1_out.at[pl.ds(ebase + j * CH, CH)], smo1[s])
        pltpu.async_copy(og2[s], g2_out.at[pl.ds(ebase + j * CH, CH)], smo2[s])

    def wait_out(s):
        pltpu.make_async_copy(og1[s], g1_out.at[pl.ds(0, CH)], smo1[s]).wait()
        pltpu.make_async_copy(og2[s], g2_out.at[pl.ds(0, CH)], smo2[s]).wait()

    def issue_deg(s, cs):
        pltpu.async_copy(ones_v, acc.at[colv[cs]], smd[s], add=True)

    def wait_deg(s, cs):
        pltpu.make_async_copy(ones_v, acc.at[colv[cs]], smd[s]).wait()

    pltpu.sync_copy(ones_hbm, ones_v)
    pltpu.sync_copy(a1_hbm, a1v)
    pltpu.sync_copy(c1_hbm, c1v)
    pltpu.sync_copy(a2_hbm, a2v)
    pltpu.sync_copy(c2_hbm, c2v)
    pltpu.sync_copy(zeros_hbm, acc.at[pl.ds(sid * RPT, RPT)])
    plsc.subcore_barrier()

    a1s = a1v[...]
    c1s = c1v[...]
    a2s = a2v[...]
    c2s = c2v[...]
    lane = lax.iota(jnp.int32, LNS)

    issue_row(0, 0)
    issue_row(1, 1)
    issue_col(0, 0)
    issue_col(1, 1)
    wait_row(0)
    issue_gr(0)
    wait_col(0)
    issue_gc(0, 0)

    @pl.loop(0, ECHUNKS // 4)
    def _quad(jj):
        for b in range(4):
            j = jj * 4 + b
            b2 = b % 2
            nb2 = (b + 1) % 2
            last_quad = jj == (ECHUNKS // 4 - 1)

            wait_gr(b2)
            wait_gc(b2, b)
            if b < 2:
                issue_row(j + 2, b2)
            else:
                @pl.when(jnp.logical_not(last_quad))
                def _pref_row():
                    issue_row(j + 2, b2)
            if b < 3:
                wait_row(nb2)
                issue_gr(nb2)
                wait_col((b + 1) % 4)
                issue_gc(nb2, (b + 1) % 4)
            else:
                @pl.when(jnp.logical_not(last_quad))
                def _next_gather():
                    wait_row(nb2)
                    issue_gr(nb2)
                    wait_col((b + 1) % 4)
                    issue_gc(nb2, (b + 1) % 4)
            if b < 2:
                @pl.when(jj > 0)
                def _drain_prev():
                    wait_out(b2)
                    wait_deg(b2, b)
            else:
                wait_out(b2)
                wait_deg(b2, b)

            @pl.loop(0, CH // LNS)
            def _grp(q):
                d2v = jnp.zeros((LNS,), jnp.float32)
                for l in range(LNS):
                    e = q * LNS + l
                    dv = br[b2][e, pl.ds(0, P)] - bc[b2][e, pl.ds(0, P)]
                    tot = jnp.sum(dv * dv, axis=0)
                    d2v = jnp.where(lane == l, tot, d2v)
                z1 = a1s * d2v + c1s
                z2 = a2s * d2v + c2s
                og1[b2][pl.ds(q * LNS, LNS)] = 1.0 / (1.0 + jnp.exp(-z1))
                og2[b2][pl.ds(q * LNS, LNS)] = 1.0 / (1.0 + jnp.exp(-z2))

            issue_out(j, b2)
            issue_deg(b2, b)
            if b < 2:
                issue_col(j + 2, (b + 2) % 4)
            else:
                @pl.when(jnp.logical_not(last_quad))
                def _pref_col():
                    issue_col(j + 2, (b + 2) % 4)

    wait_out(0)
    wait_out(1)
    wait_deg(0, 2)
    wait_deg(1, 3)
    plsc.subcore_barrier()
    pltpu.sync_copy(acc.at[pl.ds(sid * RPT, RPT)],
                    deg_out.at[cid, pl.ds(sid * RPT, RPT)])


@functools.partial(
    pl.kernel,
    out_type=jax.ShapeDtypeStruct((NC, NA, F), jnp.float32),
    mesh=_MESH,
    compiler_params=pltpu.CompilerParams(use_tc_tiling_on_sc=False, needs_layout_passes=False),
    scratch_types=(
        [pltpu.VMEM((SCH,), jnp.int32)] * 2 +
        [pltpu.VMEM((SCH,), jnp.int32)] * 4 +
        [pltpu.VMEM((SCH,), jnp.float32)] * 2 +
        [pltpu.VMEM((SCH, F), jnp.float32)] * 2 +
        [pltpu.VMEM((SCH, F), jnp.float32)] * 2 +
        [pltpu.VMEM_SHARED((NA, F), jnp.float32)] +
        [pltpu.SemaphoreType.DMA] * 12
    ),
)
def _spmm(p_hbm, row_hbm, col_hbm, g_hbm, zeros_hbm, agg_out,
          rv0, rv1, cv0, cv1, cv2, cv3, gb0, gb1, rg0, rg1, rs0, rs1, acc,
          smr0, smr1, smc0, smc1, smc2, smc3, smgt0, smgt1,
          smg0, smg1, sms0, sms1):
    cid = lax.axis_index("c")
    sid = lax.axis_index("s")
    rowv = [rv0, rv1]
    colv = [cv0, cv1, cv2, cv3]
    gbuf = [gb0, gb1]
    rg = [rg0, rg1]
    rs = [rs0, rs1]
    smr = [smr0, smr1]
    smc = [smc0, smc1, smc2, smc3]
    smgt = [smgt0, smgt1]
    smg = [smg0, smg1]
    sms = [sms0, sms1]

    def issue_row(base, j, s):
        pltpu.async_copy(row_hbm.at[pl.ds(base + j * SCH, SCH)], rowv[s], smr[s])

    def issue_col(base, j, s):
        pltpu.async_copy(col_hbm.at[pl.ds(base + j * SCH, SCH)], colv[s], smc[s])

    def issue_gate(base, j, s):
        pltpu.async_copy(g_hbm.at[pl.ds(base + j * SCH, SCH)], gbuf[s], smgt[s])

    def issue_gather(s):
        pltpu.async_copy(p_hbm.at[rowv[s]], rg[s], smg[s])

    def issue_scatter(s, cs):
        pltpu.async_copy(rs[s], acc.at[colv[cs]], sms[s], add=True)

    def wait_row(s):
        pltpu.make_async_copy(row_hbm.at[pl.ds(0, SCH)], rowv[s], smr[s]).wait()

    def wait_col(s):
        pltpu.make_async_copy(col_hbm.at[pl.ds(0, SCH)], colv[s], smc[s]).wait()

    def wait_gate(s):
        pltpu.make_async_copy(g_hbm.at[pl.ds(0, SCH)], gbuf[s], smgt[s]).wait()

    def wait_gather(s):
        pltpu.make_async_copy(p_hbm.at[rowv[s]], rg[s], smg[s]).wait()

    def wait_scatter(s, cs):
        pltpu.make_async_copy(rs[s], acc.at[colv[cs]], sms[s]).wait()

    pltpu.sync_copy(zeros_hbm, acc.at[pl.ds(sid * RPT, RPT)])
    plsc.subcore_barrier()

    def run(base, nquads):
        issue_row(base, 0, 0)
        issue_row(base, 1, 1)
        issue_col(base, 0, 0)
        issue_col(base, 1, 1)
        issue_gate(base, 0, 0)
        issue_gate(base, 1, 1)
        wait_row(0)
        issue_gather(0)

        @pl.loop(0, nquads)
        def _quad(jj):
            for b in range(4):
                j = jj * 4 + b
                b2 = b % 2
                nb2 = (b + 1) % 2
                last_quad = jj == (nquads - 1)

                wait_gather(b2)
                if b < 2:
                    issue_row(base, j + 2, b2)
                else:
                    @pl.when(jnp.logical_not(last_quad))
                    def _pref_row():
                        issue_row(base, j + 2, b2)
                if b < 3:
                    wait_row(nb2)
                    issue_gather(nb2)
                else:
                    @pl.when(jnp.logical_not(last_quad))
                    def _next_gather():
                        wait_row(nb2)
                        issue_gather(nb2)
                if b < 2:
                    @pl.when(jj > 0)
                    def _drain_scatter():
                        wait_scatter(b2, b)
                else:
                    wait_scatter(b2, b)
                wait_col(b)
                wait_gate(b2)

                @pl.loop(0, SCH // LNS)
                def _grp(q):
                    for l in range(LNS):
                        e = q * LNS + l
                        gs = plsc.load_gather(
                            gbuf[b2], [jnp.full((LNS,), e, jnp.int32)])
                        for k in range(F // LNS):
                            sl = (e, pl.ds(k * LNS, LNS))
                            rs[b2][sl] = rg[b2][sl] * gs

                issue_scatter(b2, b)
                if b < 2:
                    issue_gate(base, j + 2, b2)
                    issue_col(base, j + 2, (b + 2) % 4)
                else:
                    @pl.when(jnp.logical_not(last_quad))
                    def _pref_next():
                        issue_gate(base, j + 2, b2)
                        issue_col(base, j + 2, (b + 2) % 4)

        wait_scatter(0, 2)
        wait_scatter(1, 3)

    @pl.when(cid == 0)
    def _run_core0():
        run(sid * SE0, SQ0)

    @pl.when(cid == 1)
    def _run_core1():
        run(NS * SE0 + sid * SE1, SQ1)
    plsc.subcore_barrier()
    pltpu.sync_copy(acc.at[pl.ds(sid * RPT, RPT)],
                    agg_out.at[cid, pl.ds(sid * RPT, RPT)])


@functools.partial(
    pl.kernel,
    out_type=jax.ShapeDtypeStruct((LPAD,), jnp.float32),
    mesh=_MESH,
    compiler_params=pltpu.CompilerParams(use_tc_tiling_on_sc=False, needs_layout_passes=False),
    scratch_types=(
        [pltpu.VMEM((LCH,), jnp.int32)] * 4 +
        [pltpu.VMEM((LCH, F), jnp.float32)] * 4 +
        [pltpu.VMEM((LCH, P), jnp.float32)] * 4 +
        [pltpu.VMEM((LCH,), jnp.float32)] * 2 +
        [pltpu.VMEM((LNS,), jnp.float32)] * 3 +
        [pltpu.SemaphoreType.DMA] * 14
    ),
)
def _links(f2_hbm, coors_hbm, i0_hbm, i1_hbm, w0_hbm, w1_hbm, wb_hbm,
           res_out,
           i00, i01, i10, i11, a00, a01, a10, a11,
           q00, q01, q10, q11, ob0, ob1, w0v, w1v, wbv,
           smi00, smi01, smi10, smi11, smga0, smga1, smgb0, smgb1,
           smq00, smq01, smq10, smq11, smo0, smo1):
    cid = lax.axis_index("c")
    sid = lax.axis_index("s")
    i0v = [i00, i01]
    i1v = [i10, i11]
    a0 = [a00, a01]
    a1 = [a10, a11]
    q0 = [q00, q01]
    q1 = [q10, q11]
    obuf = [ob0, ob1]
    smi0 = [smi00, smi01]
    smi1 = [smi10, smi11]
    smga = [smga0, smga1]
    smgb = [smgb0, smgb1]
    smq0 = [smq00, smq01]
    smq1 = [smq10, smq11]
    smo = [smo0, smo1]

    def issue_idx(base, j, s):
        pltpu.async_copy(i0_hbm.at[pl.ds(base + j * LCH, LCH)], i0v[s], smi0[s])
        pltpu.async_copy(i1_hbm.at[pl.ds(base + j * LCH, LCH)], i1v[s], smi1[s])

    def wait_idx(s):
        pltpu.make_async_copy(i0_hbm.at[pl.ds(0, LCH)], i0v[s], smi0[s]).wait()
        pltpu.make_async_copy(i1_hbm.at[pl.ds(0, LCH)], i1v[s], smi1[s]).wait()

    def issue_gather(s):
        pltpu.async_copy(f2_hbm.at[i0v[s]], a0[s], smga[s])
        pltpu.async_copy(f2_hbm.at[i1v[s]], a1[s], smgb[s])
        pltpu.async_copy(coors_hbm.at[i0v[s]], q0[s], smq0[s])
        pltpu.async_copy(coors_hbm.at[i1v[s]], q1[s], smq1[s])

    def wait_gather(s):
        pltpu.make_async_copy(f2_hbm.at[i0v[s]], a0[s], smga[s]).wait()
        pltpu.make_async_copy(f2_hbm.at[i1v[s]], a1[s], smgb[s]).wait()
        pltpu.make_async_copy(coors_hbm.at[i0v[s]], q0[s], smq0[s]).wait()
        pltpu.make_async_copy(coors_hbm.at[i1v[s]], q1[s], smq1[s]).wait()

    def issue_out(base, j, s):
        pltpu.async_copy(obuf[s], res_out.at[pl.ds(base + j * LCH, LCH)], smo[s])

    def wait_out(s):
        pltpu.make_async_copy(obuf[s], res_out.at[pl.ds(0, LCH)], smo[s]).wait()

    pltpu.sync_copy(w0_hbm, w0v)
    pltpu.sync_copy(w1_hbm, w1v)
    pltpu.sync_copy(wb_hbm, wbv)
    fw0 = w0v[...]
    fw1 = w1v[...]
    fwb = wbv[...]
    lane = lax.iota(jnp.int32, LNS)

    def run(base, nchunks):
        issue_idx(base, 0, 0)
        issue_idx(base, 1, 1)
        wait_idx(0)
        issue_gather(0)

        @pl.loop(0, nchunks // 2)
        def _pair(jj):
            for b in range(2):
                j = jj * 2 + b
                nb = (b + 1) % 2
                last = jj == (nchunks // 2 - 1)

                wait_gather(b)
                if b == 0:
                    @pl.when(jnp.logical_not(last))
                    def _pref_idx():
                        issue_idx(base, j + 2, b)
                    wait_idx(nb)
                    issue_gather(nb)
                else:
                    @pl.when(jnp.logical_not(last))
                    def _pref():
                        issue_idx(base, j + 2, b)
                        wait_idx(nb)
                        issue_gather(nb)
                @pl.when(jj > 0)
                def _drain_out():
                    wait_out(b)

                @pl.loop(0, LCH // LNS)
                def _grp(q):
                    ov = jnp.zeros((LNS,), jnp.float32)
                    for l in range(LNS):
                        li = q * LNS + l
                        accv = a0[b][li, pl.ds(0, LNS)] * a1[b][li, pl.ds(0, LNS)]
                        for k in range(1, F // LNS):
                            accv = accv + (a0[b][li, pl.ds(k * LNS, LNS)] *
                                           a1[b][li, pl.ds(k * LNS, LNS)])
                        d = q0[b][li, pl.ds(0, P)] - q1[b][li, pl.ds(0, P)]
                        s = fw0 * accv + fw1 * (d * d) + fwb
                        tot = jnp.sum(s, axis=0)
                        ov = jnp.where(lane == l, tot, ov)
                    obuf[b][pl.ds(q * LNS, LNS)] = ov

                issue_out(base, j, b)

        wait_out(0)
        wait_out(1)

    @pl.when(cid == 0)
    def _run_core0():
        run(sid * LL0, LL0 // LCH)

    @pl.when(cid == 1)
    def _run_core1():
        run(NS * LL0 + sid * LL1, LL1 // LCH)


def _prescale_body(degp_ref, feats_ref, dinv_ref, p_ref):
    s = degp_ref[0] + degp_ref[1]
    dinv = lax.rsqrt(1.0 + s)
    dinv_ref[...] = dinv
    p_ref[...] = dinv[:, :1] * feats_ref[...]


def _prescale(degp, feats):
    return pl.pallas_call(
        _prescale_body,
        grid=(N // RB,),
        in_specs=[
            pl.BlockSpec((NC, RB, LNS), lambda i: (0, i, 0)),
            pl.BlockSpec((RB, F), lambda i: (i, 0)),
        ],
        out_specs=[
            pl.BlockSpec((RB, LNS), lambda i: (i, 0)),
            pl.BlockSpec((RB, F), lambda i: (i, 0)),
        ],
        out_shape=[
            jax.ShapeDtypeStruct((N, LNS), jnp.float32),
            jax.ShapeDtypeStruct((N, F), jnp.float32),
        ],
    )(degp, feats)


def _mix_body(aggp_ref, f_ref, dinv_ref, w_ref, b_ref, pb1, pw2, pb2,
              h_ref, p_ref):
    c = jnp.sum(pb1[0] * pw2[0]) + pb2[0, 0]
    g0 = jax.nn.sigmoid(c)
    di = dinv_ref[:, :1]
    pre = di * (aggp_ref[0] + aggp_ref[1]) + (g0 * di * di) * f_ref[...]
    h = jnp.dot(pre, w_ref[...], preferred_element_type=jnp.float32) + b_ref[0]
    h_ref[...] = h
    p_ref[...] = di * h


def _mix(aggp, f, dinv16, W, b, mb1, mw2, mb2):
    pspec = pl.BlockSpec((1, EDGE_MLP), lambda i: (0, 0))
    sspec = pl.BlockSpec((1, 1), lambda i: (0, 0))
    return pl.pallas_call(
        _mix_body,
        grid=(N // RB,),
        in_specs=[
            pl.BlockSpec((NC, RB, F), lambda i: (0, i, 0)),
            pl.BlockSpec((RB, F), lambda i: (i, 0)),
            pl.BlockSpec((RB, LNS), lambda i: (i, 0)),
            pl.BlockSpec((F, F), lambda i: (0, 0)),
            pl.BlockSpec((1, F), lambda i: (0, 0)),
            pspec, pspec, sspec,
        ],
        out_specs=[
            pl.BlockSpec((RB, F), lambda i: (i, 0)),
            pl.BlockSpec((RB, F), lambda i: (i, 0)),
        ],
        out_shape=[
            jax.ShapeDtypeStruct((N, F), jnp.float32),
            jax.ShapeDtypeStruct((N, F), jnp.float32),
        ],
    )(aggp, f, dinv16, W, b.reshape(1, F),
      mb1.reshape(1, EDGE_MLP), mw2.reshape(1, EDGE_MLP), mb2.reshape(1, 1))


def kernel(x, edge_index, idx, W1, b1, m1w1, m1b1, m1w2, m1b2,
           W2, b2, m2w1, m2b1, m2w2, m2b2, fcW, fcb):
    coors = x[:, :P]
    feats = x[:, P:]
    rowp = jnp.concatenate([edge_index[0], jnp.zeros((EPAD - E,), jnp.int32)])
    colp = jnp.concatenate([edge_index[1], jnp.full((EPAD - E,), N, jnp.int32)])
    i0p = jnp.concatenate([idx[0], jnp.zeros((LPAD - NLINK,), jnp.int32)])
    i1p = jnp.concatenate([idx[1], jnp.zeros((LPAD - NLINK,), jnp.int32)])
    ones16 = jnp.ones((CH, LNS), jnp.float32)
    zeros16 = jnp.zeros((RPT, LNS), jnp.float32)
    zerosF = jnp.zeros((RPT, F), jnp.float32)

    a1 = jnp.sum(m1w1[0] * m1w2[:, 0])
    c1 = m1b1 @ m1w2[:, 0] + m1b2[0]
    a2 = jnp.sum(m2w1[0] * m2w2[:, 0])
    c2 = m2b1 @ m2w2[:, 0] + m2b2[0]
    degp, g1, g2 = _edge_prep(
        rowp, colp, coors, ones16, zeros16,
        jnp.full((LNS,), a1, jnp.float32), jnp.full((LNS,), c1, jnp.float32),
        jnp.full((LNS,), a2, jnp.float32), jnp.full((LNS,), c2, jnp.float32))
    dinv16, p1 = _prescale(degp, feats)
    agg1 = _spmm(p1, rowp, colp, g1, zerosF)
    f1, p2 = _mix(agg1, feats, dinv16, W1, b1, m1b1, m1w2, m1b2)
    agg2 = _spmm(p2, rowp, colp, g2, zerosF)
    f2, _ = _mix(agg2, f1, dinv16, W2, b2, m2b1, m2w2, m2b2)

    w0 = jnp.full((LNS,), fcW[0, 0], jnp.float32)
    w1 = jnp.full((LNS,), fcW[1, 0], jnp.float32)
    wb = jnp.full((LNS,), fcb[0] / LNS, jnp.float32)
    res = _links(f2, coors, i0p, i1p, w0, w1, wb)
    return res[:NLINK].reshape(NLINK, 1)


# speedup vs baseline: 1.0810x; 1.0810x over previous
import functools

import jax
import jax.numpy as jnp
from jax import lax
from jax.experimental import pallas as pl
from jax.experimental.pallas import tpu as pltpu
from jax.experimental.pallas import tpu_sc as plsc

P = 16
F = 128
N = 10000
NA = 10240
E = 320000
NLINK = 100000
EDGE_MLP = 32

NC = 2
NS = 16
NW = NC * NS
LNS = 16

CH = 128
ECHUNKS = 80
EPT = CH * ECHUNKS
SCH = 64
SECHUNKS = EPT // SCH
SE0 = 13824
SE1 = 6656
SQ0 = SE0 // SCH // 4
SQ1 = SE1 // SCH // 4
LL0 = 4608
LL1 = 1792
EPAD = EPT * NW
RPT = NA // NS

LCH = 128
LCHUNKS = 25
LPT = LCH * LCHUNKS
LPAD = LPT * NW

GB = 2048
GBLKS = EPAD // GB
RB = 1000

_MESH = plsc.VectorSubcoreMesh(
    core_axis_name="c", subcore_axis_name="s", num_cores=NC, num_subcores=NS)


@functools.partial(
    pl.kernel,
    out_type=(
        jax.ShapeDtypeStruct((NC, NA, LNS), jnp.float32),
        jax.ShapeDtypeStruct((EPAD,), jnp.float32),
        jax.ShapeDtypeStruct((EPAD,), jnp.float32),
    ),
    mesh=_MESH,
    compiler_params=pltpu.CompilerParams(use_tc_tiling_on_sc=False, needs_layout_passes=False),
    scratch_types=(
        [pltpu.VMEM((CH,), jnp.int32)] * 2 +
        [pltpu.VMEM((CH,), jnp.int32)] * 4 +
        [pltpu.VMEM((CH, P), jnp.float32)] * 2 +
        [pltpu.VMEM((CH, P), jnp.float32)] * 2 +
        [pltpu.VMEM((CH,), jnp.float32)] * 2 +
        [pltpu.VMEM((CH,), jnp.float32)] * 2 +
        [pltpu.VMEM((CH, LNS), jnp.float32)] +
        [pltpu.VMEM((LNS,), jnp.float32)] * 4 +
        [pltpu.VMEM_SHARED((NA, LNS), jnp.float32)] +
        [pltpu.SemaphoreType.DMA] * 16
    ),
)
def _edge_prep(row_hbm, col_hbm, coors_hbm, ones_hbm, zeros_hbm,
               a1_hbm, c1_hbm, a2_hbm, c2_hbm,
               deg_out, g1_out, g2_out,
               rv0, rv1, cv0, cv1, cv2, cv3, br0, br1, bc0, bc1,
               o10, o11, o20, o21, ones_v, a1v, c1v, a2v, c2v, acc,
               smr0, smr1, smc0, smc1, smc2, smc3,
               smgr0, smgr1, smgc0, smgc1,
               smo10, smo11, smo20, smo21, smd0, smd1):
    cid = lax.axis_index("c")
    sid = lax.axis_index("s")
    wid = sid * NC + cid
    ebase = wid * EPT
    rowv = [rv0, rv1]
    colv = [cv0, cv1, cv2, cv3]
    br = [br0, br1]
    bc = [bc0, bc1]
    og1 = [o10, o11]
    og2 = [o20, o21]
    smr = [smr0, smr1]
    smc = [smc0, smc1, smc2, smc3]
    smgr = [smgr0, smgr1]
    smgc = [smgc0, smgc1]
    smo1 = [smo10, smo11]
    smo2 = [smo20, smo21]
    smd = [smd0, smd1]

    def issue_row(j, s):
        pltpu.async_copy(row_hbm.at[pl.ds(ebase + j * CH, CH)], rowv[s], smr[s])

    def issue_col(j, s):
        pltpu.async_copy(col_hbm.at[pl.ds(ebase + j * CH, CH)], colv[s], smc[s])

    def wait_row(s):
        pltpu.make_async_copy(row_hbm.at[pl.ds(0, CH)], rowv[s], smr[s]).wait()

    def wait_col(s):
        pltpu.make_async_copy(col_hbm.at[pl.ds(0, CH)], colv[s], smc[s]).wait()

    def issue_gr(s):
        pltpu.async_copy(coors_hbm.at[rowv[s]], br[s], smgr[s])

    def issue_gc(s, cs):
        pltpu.async_copy(coors_hbm.at[colv[cs]], bc[s], smgc[s])

    def wait_gr(s):
        pltpu.make_async_copy(coors_hbm.at[rowv[s]], br[s], smgr[s]).wait()

    def wait_gc(s, cs):
        pltpu.make_async_copy(coors_hbm.at[colv[cs]], bc[s], smgc[s]).wait()

    def issue_out(j, s):
        pltpu.async_copy(og1[s], g1_out.at[pl.ds(ebase + j * CH, CH)], smo1[s])
        pltpu.async_copy(og2[s], g2_out.at[pl.ds(ebase + j * CH, CH)], smo2[s])

    def wait_out(s):
        pltpu.make_async_copy(og1[s], g1_out.at[pl.ds(0, CH)], smo1[s]).wait()
        pltpu.make_async_copy(og2[s], g2_out.at[pl.ds(0, CH)], smo2[s]).wait()

    def issue_deg(s, cs):
        pltpu.async_copy(ones_v, acc.at[colv[cs]], smd[s], add=True)

    def wait_deg(s, cs):
        pltpu.make_async_copy(ones_v, acc.at[colv[cs]], smd[s]).wait()

    pltpu.sync_copy(ones_hbm, ones_v)
    pltpu.sync_copy(a1_hbm, a1v)
    pltpu.sync_copy(c1_hbm, c1v)
    pltpu.sync_copy(a2_hbm, a2v)
    pltpu.sync_copy(c2_hbm, c2v)
    pltpu.sync_copy(zeros_hbm, acc.at[pl.ds(sid * RPT, RPT)])
    plsc.subcore_barrier()

    a1s = a1v[...]
    c1s = c1v[...]
    a2s = a2v[...]
    c2s = c2v[...]
    lane = lax.iota(jnp.int32, LNS)

    issue_row(0, 0)
    issue_row(1, 1)
    issue_col(0, 0)
    issue_col(1, 1)
    wait_row(0)
    issue_gr(0)
    wait_col(0)
    issue_gc(0, 0)

    @pl.loop(0, ECHUNKS // 4)
    def _quad(jj):
        for b in range(4):
            j = jj * 4 + b
            b2 = b % 2
            nb2 = (b + 1) % 2
            last_quad = jj == (ECHUNKS // 4 - 1)

            wait_gr(b2)
            wait_gc(b2, b)
            if b < 2:
                issue_row(j + 2, b2)
            else:
                @pl.when(jnp.logical_not(last_quad))
                def _pref_row():
                    issue_row(j + 2, b2)
            if b < 3:
                wait_row(nb2)
                issue_gr(nb2)
                wait_col((b + 1) % 4)
                issue_gc(nb2, (b + 1) % 4)
            else:
                @pl.when(jnp.logical_not(last_quad))
                def _next_gather():
                    wait_row(nb2)
                    issue_gr(nb2)
                    wait_col((b + 1) % 4)
                    issue_gc(nb2, (b + 1) % 4)
            if b < 2:
                @pl.when(jj > 0)
                def _drain_prev():
                    wait_out(b2)
                    wait_deg(b2, b)
            else:
                wait_out(b2)
                wait_deg(b2, b)

            @pl.loop(0, CH // LNS)
            def _grp(q):
                d2v = jnp.zeros((LNS,), jnp.float32)
                for l in range(LNS):
                    e = q * LNS + l
                    dv = br[b2][e, pl.ds(0, P)] - bc[b2][e, pl.ds(0, P)]
                    tot = jnp.sum(dv * dv, axis=0)
                    d2v = jnp.where(lane == l, tot, d2v)
                z1 = a1s * d2v + c1s
                z2 = a2s * d2v + c2s
                og1[b2][pl.ds(q * LNS, LNS)] = 1.0 / (1.0 + jnp.exp(-z1))
                og2[b2][pl.ds(q * LNS, LNS)] = 1.0 / (1.0 + jnp.exp(-z2))

            issue_out(j, b2)
            issue_deg(b2, b)
            if b < 2:
                issue_col(j + 2, (b + 2) % 4)
            else:
                @pl.when(jnp.logical_not(last_quad))
                def _pref_col():
                    issue_col(j + 2, (b + 2) % 4)

    wait_out(0)
    wait_out(1)
    wait_deg(0, 2)
    wait_deg(1, 3)
    plsc.subcore_barrier()
    pltpu.sync_copy(acc.at[pl.ds(sid * RPT, RPT)],
                    deg_out.at[cid, pl.ds(sid * RPT, RPT)])


@functools.partial(
    pl.kernel,
    out_type=jax.ShapeDtypeStruct((NC, NA, F), jnp.float32),
    mesh=_MESH,
    compiler_params=pltpu.CompilerParams(use_tc_tiling_on_sc=False, needs_layout_passes=False),
    scratch_types=(
        [pltpu.VMEM((SCH,), jnp.int32)] * 2 +
        [pltpu.VMEM((SCH,), jnp.int32)] * 4 +
        [pltpu.VMEM((SCH,), jnp.float32)] * 2 +
        [pltpu.VMEM((SCH, F), jnp.float32)] * 2 +
        [pltpu.VMEM((SCH, F), jnp.float32)] * 2 +
        [pltpu.VMEM_SHARED((NA, F), jnp.float32)] +
        [pltpu.SemaphoreType.DMA] * 12
    ),
)
def _spmm(p_hbm, row_hbm, col_hbm, g_hbm, zeros_hbm, agg_out,
          rv0, rv1, cv0, cv1, cv2, cv3, gb0, gb1, rg0, rg1, rs0, rs1, acc,
          smr0, smr1, smc0, smc1, smc2, smc3, smgt0, smgt1,
          smg0, smg1, sms0, sms1):
    cid = lax.axis_index("c")
    sid = lax.axis_index("s")
    rowv = [rv0, rv1]
    colv = [cv0, cv1, cv2, cv3]
    gbuf = [gb0, gb1]
    rg = [rg0, rg1]
    rs = [rs0, rs1]
    smr = [smr0, smr1]
    smc = [smc0, smc1, smc2, smc3]
    smgt = [smgt0, smgt1]
    smg = [smg0, smg1]
    sms = [sms0, sms1]

    def issue_row(base, j, s):
        pltpu.async_copy(row_hbm.at[pl.ds(base + j * SCH, SCH)], rowv[s], smr[s])

    def issue_col(base, j, s):
        pltpu.async_copy(col_hbm.at[pl.ds(base + j * SCH, SCH)], colv[s], smc[s])

    def issue_gate(base, j, s):
        pltpu.async_copy(g_hbm.at[pl.ds(base + j * SCH, SCH)], gbuf[s], smgt[s])

    def issue_gather(s):
        pltpu.async_copy(p_hbm.at[rowv[s]], rg[s], smg[s])

    def issue_scatter(s, cs):
        pltpu.async_copy(rs[s], acc.at[colv[cs]], sms[s], add=True)

    def wait_row(s):
        pltpu.make_async_copy(row_hbm.at[pl.ds(0, SCH)], rowv[s], smr[s]).wait()

    def wait_col(s):
        pltpu.make_async_copy(col_hbm.at[pl.ds(0, SCH)], colv[s], smc[s]).wait()

    def wait_gate(s):
        pltpu.make_async_copy(g_hbm.at[pl.ds(0, SCH)], gbuf[s], smgt[s]).wait()

    def wait_gather(s):
        pltpu.make_async_copy(p_hbm.at[rowv[s]], rg[s], smg[s]).wait()

    def wait_scatter(s, cs):
        pltpu.make_async_copy(rs[s], acc.at[colv[cs]], sms[s]).wait()

    pltpu.sync_copy(zeros_hbm, acc.at[pl.ds(sid * RPT, RPT)])
    plsc.subcore_barrier()

    def run(base, nquads):
        issue_row(base, 0, 0)
        issue_row(base, 1, 1)
        issue_col(base, 0, 0)
        issue_col(base, 1, 1)
        issue_gate(base, 0, 0)
        issue_gate(base, 1, 1)
        wait_row(0)
        issue_gather(0)

        @pl.loop(0, nquads)
        def _quad(jj):
            for b in range(4):
                j = jj * 4 + b
                b2 = b % 2
                nb2 = (b + 1) % 2
                last_quad = jj == (nquads - 1)

                wait_gather(b2)
                if b < 2:
                    issue_row(base, j + 2, b2)
                else:
                    @pl.when(jnp.logical_not(last_quad))
                    def _pref_row():
                        issue_row(base, j + 2, b2)
                if b < 3:
                    wait_row(nb2)
                    issue_gather(nb2)
                else:
                    @pl.when(jnp.logical_not(last_quad))
                    def _next_gather():
                        wait_row(nb2)
                        issue_gather(nb2)
                if b < 2:
                    @pl.when(jj > 0)
                    def _drain_scatter():
                        wait_scatter(b2, b)
                else:
                    wait_scatter(b2, b)
                wait_col(b)
                wait_gate(b2)

                @pl.loop(0, SCH // LNS)
                def _grp(q):
                    for l in range(LNS):
                        e = q * LNS + l
                        gs = plsc.load_gather(
                            gbuf[b2], [jnp.full((LNS,), e, jnp.int32)])
                        for k in range(F // LNS):
                            sl = (e, pl.ds(k * LNS, LNS))
                            rs[b2][sl] = rg[b2][sl] * gs

                issue_scatter(b2, b)
                if b < 2:
                    issue_gate(base, j + 2, b2)
                    issue_col(base, j + 2, (b + 2) % 4)
                else:
                    @pl.when(jnp.logical_not(last_quad))
                    def _pref_next():
                        issue_gate(base, j + 2, b2)
                        issue_col(base, j + 2, (b + 2) % 4)

        wait_scatter(0, 2)
        wait_scatter(1, 3)

    @pl.when(cid == 0)
    def _run_core0():
        run(sid * SE0, SQ0)

    @pl.when(cid == 1)
    def _run_core1():
        run(NS * SE0 + sid * SE1, SQ1)
    plsc.subcore_barrier()
    pltpu.sync_copy(acc.at[pl.ds(sid * RPT, RPT)],
                    agg_out.at[cid, pl.ds(sid * RPT, RPT)])


@functools.partial(
    pl.kernel,
    out_type=jax.ShapeDtypeStruct((LPAD,), jnp.float32),
    mesh=_MESH,
    compiler_params=pltpu.CompilerParams(use_tc_tiling_on_sc=False, needs_layout_passes=False),
    scratch_types=(
        [pltpu.VMEM((LCH,), jnp.int32)] * 4 +
        [pltpu.VMEM((LCH, F), jnp.float32)] * 4 +
        [pltpu.VMEM((LCH, P), jnp.float32)] * 4 +
        [pltpu.VMEM((LCH,), jnp.float32)] * 2 +
        [pltpu.VMEM((LNS,), jnp.float32)] * 3 +
        [pltpu.SemaphoreType.DMA] * 14
    ),
)
def _links(f2_hbm, coors_hbm, i0_hbm, i1_hbm, w0_hbm, w1_hbm, wb_hbm,
           res_out,
           i00, i01, i10, i11, a00, a01, a10, a11,
           q00, q01, q10, q11, ob0, ob1, w0v, w1v, wbv,
           smi00, smi01, smi10, smi11, smga0, smga1, smgb0, smgb1,
           smq00, smq01, smq10, smq11, smo0, smo1):
    cid = lax.axis_index("c")
    sid = lax.axis_index("s")
    i0v = [i00, i01]
    i1v = [i10, i11]
    a0 = [a00, a01]
    a1 = [a10, a11]
    q0 = [q00, q01]
    q1 = [q10, q11]
    obuf = [ob0, ob1]
    smi0 = [smi00, smi01]
    smi1 = [smi10, smi11]
    smga = [smga0, smga1]
    smgb = [smgb0, smgb1]
    smq0 = [smq00, smq01]
    smq1 = [smq10, smq11]
    smo = [smo0, smo1]

    def issue_idx(base, j, s):
        pltpu.async_copy(i0_hbm.at[pl.ds(base + j * LCH, LCH)], i0v[s], smi0[s])
        pltpu.async_copy(i1_hbm.at[pl.ds(base + j * LCH, LCH)], i1v[s], smi1[s])

    def wait_idx(s):
        pltpu.make_async_copy(i0_hbm.at[pl.ds(0, LCH)], i0v[s], smi0[s]).wait()
        pltpu.make_async_copy(i1_hbm.at[pl.ds(0, LCH)], i1v[s], smi1[s]).wait()

    def issue_gather(s):
        pltpu.async_copy(f2_hbm.at[i0v[s]], a0[s], smga[s])
        pltpu.async_copy(f2_hbm.at[i1v[s]], a1[s], smgb[s])
        pltpu.async_copy(coors_hbm.at[i0v[s]], q0[s], smq0[s])
        pltpu.async_copy(coors_hbm.at[i1v[s]], q1[s], smq1[s])

    def wait_gather(s):
        pltpu.make_async_copy(f2_hbm.at[i0v[s]], a0[s], smga[s]).wait()
        pltpu.make_async_copy(f2_hbm.at[i1v[s]], a1[s], smgb[s]).wait()
        pltpu.make_async_copy(coors_hbm.at[i0v[s]], q0[s], smq0[s]).wait()
        pltpu.make_async_copy(coors_hbm.at[i1v[s]], q1[s], smq1[s]).wait()

    def issue_out(base, j, s):
        pltpu.async_copy(obuf[s], res_out.at[pl.ds(base + j * LCH, LCH)], smo[s])

    def wait_out(s):
        pltpu.make_async_copy(obuf[s], res_out.at[pl.ds(0, LCH)], smo[s]).wait()

    pltpu.sync_copy(w0_hbm, w0v)
    pltpu.sync_copy(w1_hbm, w1v)
    pltpu.sync_copy(wb_hbm, wbv)
    fw0 = w0v[...]
    fw1 = w1v[...]
    fwb = wbv[...]
    lane = lax.iota(jnp.int32, LNS)

    def run(base, nchunks):
        issue_idx(base, 0, 0)
        issue_idx(base, 1, 1)
        wait_idx(0)
        issue_gather(0)

        @pl.loop(0, nchunks // 2)
        def _pair(jj):
            for b in range(2):
                j = jj * 2 + b
                nb = (b + 1) % 2
                last = jj == (nchunks // 2 - 1)

                wait_gather(b)
                if b == 0:
                    @pl.when(jnp.logical_not(last))
                    def _pref_idx():
                        issue_idx(base, j + 2, b)
                    wait_idx(nb)
                    issue_gather(nb)
                else:
                    @pl.when(jnp.logical_not(last))
                    def _pref():
                        issue_idx(base, j + 2, b)
                        wait_idx(nb)
                        issue_gather(nb)
                @pl.when(jj > 0)
                def _drain_out():
                    wait_out(b)

                @pl.loop(0, LCH // LNS)
                def _grp(q):
                    ov = jnp.zeros((LNS,), jnp.float32)
                    for l in range(LNS):
                        li = q * LNS + l
                        accv = a0[b][li, pl.ds(0, LNS)] * a1[b][li, pl.ds(0, LNS)]
                        for k in range(1, F // LNS):
                            accv = accv + (a0[b][li, pl.ds(k * LNS, LNS)] *
                                           a1[b][li, pl.ds(k * LNS, LNS)])
                        d = q0[b][li, pl.ds(0, P)] - q1[b][li, pl.ds(0, P)]
                        s = fw0 * accv + fw1 * (d * d) + fwb
                        tot = jnp.sum(s, axis=0)
                        ov = jnp.where(lane == l, tot, ov)
                    obuf[b][pl.ds(q * LNS, LNS)] = ov

                issue_out(base, j, b)

        wait_out(0)
        wait_out(1)

    @pl.when(cid == 0)
    def _run_core0():
        run(sid * LL0, LL0 // LCH)

    @pl.when(cid == 1)
    def _run_core1():
        run(NS * LL0 + sid * LL1, LL1 // LCH)


def _prescale_body(degp_ref, feats_ref, dinv_ref, p_ref):
    s = degp_ref[0] + degp_ref[1]
    dinv = lax.rsqrt(1.0 + s)
    dinv_ref[...] = dinv
    p_ref[...] = dinv[:, :1] * feats_ref[...]


def _prescale(degp, feats):
    return pl.pallas_call(
        _prescale_body,
        grid=(N // RB,),
        in_specs=[
            pl.BlockSpec((NC, RB, LNS), lambda i: (0, i, 0)),
            pl.BlockSpec((RB, F), lambda i: (i, 0)),
        ],
        out_specs=[
            pl.BlockSpec((RB, LNS), lambda i: (i, 0)),
            pl.BlockSpec((RB, F), lambda i: (i, 0)),
        ],
        out_shape=[
            jax.ShapeDtypeStruct((N, LNS), jnp.float32),
            jax.ShapeDtypeStruct((N, F), jnp.float32),
        ],
    )(degp, feats)


def _mix_body(aggp_ref, f_ref, dinv_ref, w_ref, b_ref, pb1, pw2, pb2,
              h_ref, p_ref):
    c = jnp.sum(pb1[0] * pw2[0]) + pb2[0, 0]
    g0 = jax.nn.sigmoid(c)
    di = dinv_ref[:, :1]
    pre = di * (aggp_ref[0] + aggp_ref[1]) + (g0 * di * di) * f_ref[...]
    h = jnp.dot(pre, w_ref[...], preferred_element_type=jnp.float32) + b_ref[0]
    h_ref[...] = h
    p_ref[...] = di * h


def _mix(aggp, f, dinv16, W, b, mb1, mw2, mb2):
    pspec = pl.BlockSpec((1, EDGE_MLP), lambda i: (0, 0))
    sspec = pl.BlockSpec((1, 1), lambda i: (0, 0))
    return pl.pallas_call(
        _mix_body,
        grid=(N // RB,),
        in_specs=[
            pl.BlockSpec((NC, RB, F), lambda i: (0, i, 0)),
            pl.BlockSpec((RB, F), lambda i: (i, 0)),
            pl.BlockSpec((RB, LNS), lambda i: (i, 0)),
            pl.BlockSpec((F, F), lambda i: (0, 0)),
            pl.BlockSpec((1, F), lambda i: (0, 0)),
            pspec, pspec, sspec,
        ],
        out_specs=[
            pl.BlockSpec((RB, F), lambda i: (i, 0)),
            pl.BlockSpec((RB, F), lambda i: (i, 0)),
        ],
        out_shape=[
            jax.ShapeDtypeStruct((N, F), jnp.float32),
            jax.ShapeDtypeStruct((N, F), jnp.float32),
        ],
    )(aggp, f, dinv16, W, b.reshape(1, F),
      mb1.reshape(1, EDGE_MLP), mw2.reshape(1, EDGE_MLP), mb2.reshape(1, 1))


def kernel(x, edge_index, idx, W1, b1, m1w1, m1b1, m1w2, m1b2,
           W2, b2, m2w1, m2b1, m2w2, m2b2, fcW, fcb):
    coors = x[:, :P]
    feats = x[:, P:]
    rowp = jnp.concatenate([edge_index[0], jnp.zeros((EPAD - E,), jnp.int32)])
    colp = jnp.concatenate([edge_index[1], jnp.full((EPAD - E,), N, jnp.int32)])
    i0p = jnp.concatenate([idx[0], jnp.zeros((LPAD - NLINK,), jnp.int32)])
    i1p = jnp.concatenate([idx[1], jnp.zeros((LPAD - NLINK,), jnp.int32)])
    ones16 = jnp.ones((CH, LNS), jnp.float32)
    zeros16 = jnp.zeros((RPT, LNS), jnp.float32)
    zerosF = jnp.zeros((RPT, F), jnp.float32)

    a1 = jnp.sum(m1w1[0] * m1w2[:, 0])
    c1 = m1b1 @ m1w2[:, 0] + m1b2[0]
    a2 = jnp.sum(m2w1[0] * m2w2[:, 0])
    c2 = m2b1 @ m2w2[:, 0] + m2b2[0]
    degp, g1, g2 = _edge_prep(
        rowp, colp, coors, ones16, zeros16,
        jnp.full((LNS,), a1, jnp.float32), jnp.full((LNS,), c1, jnp.float32),
        jnp.full((LNS,), a2, jnp.float32), jnp.full((LNS,), c2, jnp.float32))
    dinv16, p1 = _prescale(degp, feats)
    agg1 = _spmm(p1, rowp, colp, g1, zerosF)
    f1, p2 = _mix(agg1, feats, dinv16, W1, b1, m1b1, m1w2, m1b2)
    agg2 = _spmm(p2, rowp, colp, g2, zerosF)
    f2, _ = _mix(agg2, f1, dinv16, W2, b2, m2b1, m2w2, m2b2)

    w0 = jnp.full((LNS,), fcW[0, 0], jnp.float32)
    w1 = jnp.full((LNS,), fcW[1, 0], jnp.float32)
    wb = jnp.full((LNS,), fcb[0] / LNS, jnp.float32)
    res = _links(f2, coors, i0p, i1p, w0, w1, wb)
    return res[:NLINK].reshape(NLINK, 1)
